# Initial kernel scaffold; baseline (speedup 1.0000x reference)
import functools

import jax
import jax.numpy as jnp
from jax.experimental import pallas as pl
from jax.experimental.pallas import tpu as pltpu

B, S, D, DFF, E, K = 1, 2048, 768, 2048, 8, 2
N = B * S
T_TILES = 4
TILE_N = N // T_TILES


def _router_body(x_ref, wr_ref, gates_ref):
    logits = jax.lax.dot_general(
        x_ref[...], wr_ref[...], (((1,), (1,)), ((), ())),
        preferred_element_type=jnp.float32,
        precision=jax.lax.Precision.HIGHEST)
    m = jnp.max(logits, axis=1, keepdims=True)
    p = jnp.exp(logits - m)
    probs = p / jnp.sum(p, axis=1, keepdims=True)

    lane = jax.lax.broadcasted_iota(jnp.int32, (N, E), 1)
    m1 = jnp.max(probs, axis=1, keepdims=True)
    i1 = jnp.min(jnp.where(probs == m1, lane, E), axis=1, keepdims=True)
    probs2 = jnp.where(lane == i1, -1.0, probs)
    m2 = jnp.max(probs2, axis=1, keepdims=True)
    i2 = jnp.min(jnp.where(probs2 == m2, lane, E), axis=1, keepdims=True)

    denom = m1 + m2 + 1e-9
    g1 = m1 / denom
    g2 = m2 / denom
    gates_ref[...] = (jnp.where(lane == i1, g1, 0.0)
                      + jnp.where(lane == i2, g2, 0.0))


def _ffn_body(gates_ref, x_ref, w1_ref, b1_ref, w2_ref, b2_ref, out_ref):
    e = pl.program_id(0)
    t = pl.program_id(1)
    xb = x_ref[...]
    h = jax.lax.dot_general(
        xb, w1_ref[0], (((1,), (0,)), ((), ())),
        preferred_element_type=jnp.float32)
    h = h + b1_ref[...]
    h = jax.nn.gelu(h, approximate=False)
    y = jax.lax.dot_general(
        h.astype(jnp.bfloat16), w2_ref[0], (((1,), (0,)), ((), ())),
        preferred_element_type=jnp.float32)
    y = y + b2_ref[...]
    g = jax.lax.dynamic_slice(gates_ref[...], (0, e), (TILE_N, 1))
    contrib = g * y
    sl = pl.ds(t * TILE_N, TILE_N)

    @pl.when(e == 0)
    def _():
        out_ref[sl, :] = contrib

    @pl.when(e != 0)
    def _():
        out_ref[sl, :] = out_ref[sl, :] + contrib


@jax.jit
def kernel(x, Wr, W1, b1, W2, b2):
    x_flat = x.reshape(N, D)
    gates = pl.pallas_call(
        _router_body,
        out_shape=jax.ShapeDtypeStruct((N, E), jnp.float32),
        in_specs=[pl.BlockSpec((N, D), lambda: (0, 0)),
                  pl.BlockSpec((E, D), lambda: (0, 0))],
        out_specs=pl.BlockSpec((N, E), lambda: (0, 0)),
    )(x_flat, Wr)

    xb = x_flat.astype(jnp.bfloat16)
    w1b = W1.astype(jnp.bfloat16)
    w2b = W2.astype(jnp.bfloat16)

    out = pl.pallas_call(
        _ffn_body,
        grid=(E, T_TILES),
        out_shape=jax.ShapeDtypeStruct((N, D), jnp.float32),
        in_specs=[
            pl.BlockSpec((TILE_N, E), lambda e, t: (t, 0)),
            pl.BlockSpec((TILE_N, D), lambda e, t: (t, 0)),
            pl.BlockSpec((1, D, DFF), lambda e, t: (e, 0, 0)),
            pl.BlockSpec((1, DFF), lambda e, t: (e, 0)),
            pl.BlockSpec((1, DFF, D), lambda e, t: (e, 0, 0)),
            pl.BlockSpec((1, D), lambda e, t: (e, 0)),
        ],
        out_specs=pl.BlockSpec((N, D), lambda e, t: (0, 0)),
        compiler_params=pltpu.CompilerParams(
            dimension_semantics=("arbitrary", "arbitrary")),
    )(gates, xb, w1b, b1, w2b, b2)
    return out.reshape(B, S, D)


# speedup vs baseline: 2.6803x; 2.6803x over previous
import functools

import jax
import jax.numpy as jnp
from jax.experimental import pallas as pl
from jax.experimental.pallas import tpu as pltpu

B, S, D, DFF, E, K = 1, 2048, 768, 2048, 8, 2
N = B * S
T_TILES = 4
TILE_N = N // T_TILES


def _router_body(x_ref, wr_ref, gates_ref):
    logits = jax.lax.dot_general(
        x_ref[...], wr_ref[...], (((1,), (1,)), ((), ())),
        preferred_element_type=jnp.float32,
        precision=jax.lax.Precision.DEFAULT)
    m = jnp.max(logits, axis=1, keepdims=True)
    p = jnp.exp(logits - m)
    probs = p / jnp.sum(p, axis=1, keepdims=True)

    lane = jax.lax.broadcasted_iota(jnp.int32, (N, E), 1)
    m1 = jnp.max(probs, axis=1, keepdims=True)
    i1 = jnp.min(jnp.where(probs == m1, lane, E), axis=1, keepdims=True)
    probs2 = jnp.where(lane == i1, -1.0, probs)
    m2 = jnp.max(probs2, axis=1, keepdims=True)
    i2 = jnp.min(jnp.where(probs2 == m2, lane, E), axis=1, keepdims=True)

    denom = m1 + m2 + 1e-9
    g1 = m1 / denom
    g2 = m2 / denom
    gates_ref[...] = (jnp.where(lane == i1, g1, 0.0)
                      + jnp.where(lane == i2, g2, 0.0))


def _ffn_body(gates_ref, x_ref, w1_ref, b1_ref, w2_ref, b2_ref, out_ref):
    e = pl.program_id(0)
    t = pl.program_id(1)
    xb = x_ref[...]
    h = jax.lax.dot_general(
        xb, w1_ref[0], (((1,), (0,)), ((), ())),
        preferred_element_type=jnp.float32)
    h = h + b1_ref[0]
    h = 0.5 * h * (1.0 + jax.lax.erf(h * 0.7071067811865476))
    y = jax.lax.dot_general(
        h.astype(jnp.bfloat16), w2_ref[0], (((1,), (0,)), ((), ())),
        preferred_element_type=jnp.float32)
    y = y + b2_ref[0]
    lane = jax.lax.broadcasted_iota(jnp.int32, (TILE_N, E), 1)
    g = jnp.sum(jnp.where(lane == e, gates_ref[...], 0.0), axis=1,
                keepdims=True)
    contrib = g * y
    sl = pl.ds(t * TILE_N, TILE_N)

    @pl.when(e == 0)
    def _():
        out_ref[sl, :] = contrib

    @pl.when(e != 0)
    def _():
        out_ref[sl, :] = out_ref[sl, :] + contrib


@jax.jit
def kernel(x, Wr, W1, b1, W2, b2):
    x_flat = x.reshape(N, D)
    gates = pl.pallas_call(
        _router_body,
        out_shape=jax.ShapeDtypeStruct((N, E), jnp.float32),
        in_specs=[pl.BlockSpec((N, D), lambda: (0, 0)),
                  pl.BlockSpec((E, D), lambda: (0, 0))],
        out_specs=pl.BlockSpec((N, E), lambda: (0, 0)),
    )(x_flat, Wr)

    xb = x_flat.astype(jnp.bfloat16)
    w1b = W1.astype(jnp.bfloat16)
    w2b = W2.astype(jnp.bfloat16)

    out = pl.pallas_call(
        _ffn_body,
        grid=(E, T_TILES),
        out_shape=jax.ShapeDtypeStruct((N, D), jnp.float32),
        in_specs=[
            pl.BlockSpec((TILE_N, E), lambda e, t: (t, 0)),
            pl.BlockSpec((TILE_N, D), lambda e, t: (t, 0)),
            pl.BlockSpec((1, D, DFF), lambda e, t: (e, 0, 0)),
            pl.BlockSpec((1, 1, DFF), lambda e, t: (e, 0, 0)),
            pl.BlockSpec((1, DFF, D), lambda e, t: (e, 0, 0)),
            pl.BlockSpec((1, 1, D), lambda e, t: (e, 0, 0)),
        ],
        out_specs=pl.BlockSpec((N, D), lambda e, t: (0, 0)),
        compiler_params=pltpu.CompilerParams(
            dimension_semantics=("arbitrary", "arbitrary")),
    )(gates, xb, w1b, b1.reshape(E, 1, DFF), w2b, b2.reshape(E, 1, D))
    return out.reshape(B, S, D)
